# Initial kernel scaffold; baseline (speedup 1.0000x reference)
import jax
import jax.numpy as jnp
from jax.experimental import pallas as pl


def kernel(node_type, edge_index, edge_dist, graph_ids, fts, embed, W1, Wc1, bc1, Wc2, bc2, W2, b2, W3, b3, Wd1, bd1, Wd2, bd2, F1, bf1, F2, bf2, F3, bf3, F4, bf4):
    raise NotImplementedError("write your pallas kernel here")


# speedup vs baseline: 1.6555x; 1.6555x over previous
import functools

import jax
import jax.numpy as jnp
import numpy as np
from jax import lax
from jax.experimental import pallas as pl
from jax.experimental.pallas import tpu as pltpu
from jax.experimental.pallas import tpu_sc as plsc

DIM = 64
DIMP = 128
CUTOFF = 5.0
N_CENTERS = 5
_centers_np = np.linspace(0.0, CUTOFF, N_CENTERS)
GAP = float(_centers_np[1] - _centers_np[0])
N_NODES = 10000
N_EDGES = 320000
N_GRAPHS = 9
B = 64
N_CONV = 3
LOG2 = float(np.log(2.0))

NODE_CH = 2000
EDGE_CH = 2500

SC_NC = 2
SC_NS = 16
SC_NW = SC_NC * SC_NS
SC_C = 128
SC_CHUNKS = N_EDGES // SC_C
SC_FULL = SC_CHUNKS // SC_NW
SC_EXTRA = SC_CHUNKS - SC_FULL * SC_NW
ROWS_PER_TILE = 624
ROWS_TAIL = N_NODES - ROWS_PER_TILE * SC_NS


def _sp05(x):
    z = 0.5 * x
    return 2.0 * (jnp.maximum(z, 0.0) + jnp.log1p(jnp.exp(-jnp.abs(z))))


def _softplus(x):
    return jnp.maximum(x, 0.0) + jnp.log1p(jnp.exp(-jnp.abs(x)))


def _embed_body(nt_ref, emb_ref, w10_ref, node_ref, nn_ref):
    ntc = nt_ref[0, 0]
    oh = (ntc == lax.broadcasted_iota(jnp.int32, (1, 100), 1)).astype(jnp.float32)
    node = jnp.dot(oh, emb_ref[...], preferred_element_type=jnp.float32)
    node_ref[0] = node
    nn = jnp.dot(node, w10_ref[...], preferred_element_type=jnp.float32)
    nn_ref[0] = jnp.concatenate([nn, jnp.zeros_like(nn)], axis=-1)


def _embed_call(nt4, emb, w10):
    n_ch = N_NODES // NODE_CH
    return pl.pallas_call(
        _embed_body,
        grid=(N_GRAPHS, n_ch),
        in_specs=[
            pl.BlockSpec((1, 1, NODE_CH, 1), lambda g, c: (g, c, 0, 0)),
            pl.BlockSpec((100, DIM), lambda g, c: (0, 0)),
            pl.BlockSpec((DIM, DIM), lambda g, c: (0, 0)),
        ],
        out_specs=[
            pl.BlockSpec((1, NODE_CH, DIM), lambda g, c: (g, c, 0)),
            pl.BlockSpec((1, NODE_CH, DIMP), lambda g, c: (g, c, 0)),
        ],
        out_shape=[
            jax.ShapeDtypeStruct((N_GRAPHS, N_NODES, DIM), jnp.float32),
            jax.ShapeDtypeStruct((N_GRAPHS, N_NODES, DIMP), jnp.float32),
        ],
    )(nt4, emb, w10)


def _h_body(d_ref, wc1_ref, bc1_ref, wc2_ref, bc2_ref, h_ref):
    d = d_ref[0, 0]
    cen = GAP * lax.broadcasted_iota(jnp.int32, (1, N_CENTERS), 1).astype(jnp.float32)
    rbf = jnp.exp((-1.0 / GAP) * (d - cen) ** 2)
    u = jnp.dot(rbf, wc1_ref[...], preferred_element_type=jnp.float32) + bc1_ref[...]
    u = _sp05(u)
    h = jnp.dot(u, wc2_ref[...], preferred_element_type=jnp.float32) + bc2_ref[...]
    h_ref[0, 0] = h


def _h_call(dist4, wc1, bc1, wc2, bc2):
    e_ch = N_EDGES // EDGE_CH
    return pl.pallas_call(
        _h_body,
        grid=(N_GRAPHS, e_ch),
        in_specs=[
            pl.BlockSpec((1, 1, EDGE_CH, 1), lambda g, c: (g, c, 0, 0)),
            pl.BlockSpec((N_CENTERS, DIM), lambda g, c: (0, 0)),
            pl.BlockSpec((DIM,), lambda g, c: (0,)),
            pl.BlockSpec((DIM, DIM), lambda g, c: (0, 0)),
            pl.BlockSpec((DIM,), lambda g, c: (0,)),
        ],
        out_specs=pl.BlockSpec((1, 1, EDGE_CH, DIM), lambda g, c: (g, c, 0, 0)),
        out_shape=jax.ShapeDtypeStruct((N_GRAPHS, e_ch, EDGE_CH, DIM), jnp.float32),
    )(dist4, wc1, bc1, wc2, bc2)


def _upd_body_mid(node_ref, agg_ref, w2_ref, b2_ref, w3_ref, b3_ref, w1n_ref,
                  node_out, nn_out):
    a = (agg_ref[0, 0] + agg_ref[0, 1])[:, :DIM]
    t = _sp05(jnp.dot(a, w2_ref[...], preferred_element_type=jnp.float32) + b2_ref[...])
    nd = node_ref[0] + jnp.dot(t, w3_ref[...], preferred_element_type=jnp.float32) + b3_ref[...]
    node_out[0] = nd
    nn = jnp.dot(nd, w1n_ref[...], preferred_element_type=jnp.float32)
    nn_out[0] = jnp.concatenate([nn, jnp.zeros_like(nn)], axis=-1)


def _upd_body_last(node_ref, agg_ref, w2_ref, b2_ref, w3_ref, b3_ref, node_out):
    a = (agg_ref[0, 0] + agg_ref[0, 1])[:, :DIM]
    t = _sp05(jnp.dot(a, w2_ref[...], preferred_element_type=jnp.float32) + b2_ref[...])
    node_out[0] = node_ref[0] + jnp.dot(t, w3_ref[...], preferred_element_type=jnp.float32) + b3_ref[...]


def _upd_call(node, agg4, w2, b2, w3, b3, w1n):
    n_ch = N_NODES // NODE_CH
    in_specs = [
        pl.BlockSpec((1, NODE_CH, DIM), lambda g, c: (g, c, 0)),
        pl.BlockSpec((1, 2, NODE_CH, DIMP), lambda g, c: (g, 0, c, 0)),
        pl.BlockSpec((DIM, DIM), lambda g, c: (0, 0)),
        pl.BlockSpec((DIM,), lambda g, c: (0,)),
        pl.BlockSpec((DIM, DIM), lambda g, c: (0, 0)),
        pl.BlockSpec((DIM,), lambda g, c: (0,)),
    ]
    node_spec = pl.BlockSpec((1, NODE_CH, DIM), lambda g, c: (g, c, 0))
    node_shape = jax.ShapeDtypeStruct((N_GRAPHS, N_NODES, DIM), jnp.float32)
    if w1n is None:
        return pl.pallas_call(
            _upd_body_last,
            grid=(N_GRAPHS, n_ch),
            in_specs=in_specs,
            out_specs=node_spec,
            out_shape=node_shape,
        )(node, agg4, w2, b2, w3, b3)
    return pl.pallas_call(
        _upd_body_mid,
        grid=(N_GRAPHS, n_ch),
        in_specs=in_specs + [pl.BlockSpec((DIM, DIM), lambda g, c: (0, 0))],
        out_specs=[node_spec,
                   pl.BlockSpec((1, NODE_CH, DIMP), lambda g, c: (g, c, 0))],
        out_shape=[node_shape,
                   jax.ShapeDtypeStruct((N_GRAPHS, N_NODES, DIMP), jnp.float32)],
    )(node, agg4, w2, b2, w3, b3, w1n)


def _readout_body(node_ref, gid_ref, wd1_ref, bd1_ref, wd2_ref, bd2_ref, out_ref):
    x = node_ref[0]
    a = jnp.dot(x, wd1_ref[...], preferred_element_type=jnp.float32) + bd1_ref[...]
    atom = _softplus(a) - LOG2
    res = jnp.dot(atom, wd2_ref[...], preferred_element_type=jnp.float32) + bd2_ref[...]
    gid = gid_ref[0]
    mask = (gid == lax.broadcasted_iota(jnp.int32, (1, B), 1)).astype(jnp.float32)
    contrib = jnp.sum(mask * res, axis=0, keepdims=True)

    @pl.when(pl.program_id(1) == 0)
    def _():
        out_ref[...] = jnp.zeros_like(out_ref)

    out_ref[...] += contrib


def _readout_call(node, gid3, wd1, bd1, wd2, bd2):
    n_ch = N_NODES // NODE_CH
    return pl.pallas_call(
        _readout_body,
        grid=(N_GRAPHS, n_ch),
        in_specs=[
            pl.BlockSpec((1, NODE_CH, DIM), lambda g, c: (g, c, 0)),
            pl.BlockSpec((1, NODE_CH, 1), lambda g, c: (c, 0, 0)),
            pl.BlockSpec((DIM, DIM), lambda g, c: (0, 0)),
            pl.BlockSpec((DIM,), lambda g, c: (0,)),
            pl.BlockSpec((DIM, 1), lambda g, c: (0, 0)),
            pl.BlockSpec((1,), lambda g, c: (0,)),
        ],
        out_specs=pl.BlockSpec((1, 1, B), lambda g, c: (g, 0, 0)),
        out_shape=jax.ShapeDtypeStruct((N_GRAPHS, 1, B), jnp.float32),
    )(node, gid3, wd1, bd1, wd2, bd2)


def _mlp_body(dense_ref, f1_ref, b1_ref, f2_ref, b2_ref, f3_ref, b3_ref,
              f4_ref, b4_ref, out_ref):
    h = jnp.maximum(jnp.dot(dense_ref[...], f1_ref[...], preferred_element_type=jnp.float32) + b1_ref[...], 0.0)
    h = jnp.maximum(jnp.dot(h, f2_ref[...], preferred_element_type=jnp.float32) + b2_ref[...], 0.0)
    h = jnp.maximum(jnp.dot(h, f3_ref[...], preferred_element_type=jnp.float32) + b3_ref[...], 0.0)
    out_ref[...] = jnp.dot(h, f4_ref[...], preferred_element_type=jnp.float32) + b4_ref[...]


def _mlp_call(dense, f1, b1, f2, b2, f3, b3, f4, b4):
    return pl.pallas_call(
        _mlp_body,
        out_shape=jax.ShapeDtypeStruct((B, 1), jnp.float32),
    )(dense, f1, b1, f2, b2, f3, b3, f4, b4)


def _sc_body(nn_hbm, h_hbm, src_hbm, dst_hbm, out_hbm,
             srcv, dstv, nnv, hv, zbuf, aggs, sem):
    cid = lax.axis_index("c")
    sid = lax.axis_index("s")
    wid = sid * SC_NC + cid
    r0 = sid * ROWS_PER_TILE
    rem = ROWS_PER_TILE % SC_C
    last = sid == SC_NS - 1

    def _zb(r, carry):
        for k in range(DIMP // 16):
            zbuf[r, pl.ds(k * 16, 16)] = jnp.zeros((16,), jnp.float32)
        return carry

    lax.fori_loop(0, SC_C, _zb, 0)

    def _do_chunk(g, c):
        base = c * SC_C
        gb = g * N_EDGES + base
        pltpu.sync_copy(src_hbm.at[pl.ds(gb, SC_C)], srcv)
        pltpu.sync_copy(dst_hbm.at[pl.ds(base, SC_C)], dstv)
        pltpu.async_copy(nn_hbm.at[srcv], nnv, sem).wait()
        pltpu.sync_copy(h_hbm.at[pl.ds(gb, SC_C)], hv)

        def _mul(r, carry):
            for k in range(DIM // 16):
                sl = pl.ds(k * 16, 16)
                nnv[r, sl] = nnv[r, sl] * hv[r, sl]
            return carry

        lax.fori_loop(0, SC_C, _mul, 0)
        pltpu.sync_copy(nnv, aggs.at[dstv], add=True)

    def _per_graph(g, carry):
        for t in range(ROWS_PER_TILE // SC_C):
            pltpu.sync_copy(zbuf, aggs.at[pl.ds(r0 + t * SC_C, SC_C)])
        if rem:
            pltpu.sync_copy(zbuf.at[pl.ds(0, rem)],
                            aggs.at[pl.ds(r0 + (ROWS_PER_TILE // SC_C) * SC_C, rem)])

        @pl.when(last)
        def _():
            pltpu.sync_copy(zbuf.at[pl.ds(0, ROWS_TAIL)],
                            aggs.at[pl.ds(SC_NS * ROWS_PER_TILE, ROWS_TAIL)])

        plsc.subcore_barrier()

        def _round(j, inner):
            _do_chunk(g, j * SC_NW + wid)
            return inner

        lax.fori_loop(0, SC_FULL, _round, 0)

        @pl.when(wid < SC_EXTRA)
        def _():
            _do_chunk(g, SC_FULL * SC_NW + wid)

        plsc.subcore_barrier()

        ob = (g * SC_NC + cid) * N_NODES + r0
        for t in range(ROWS_PER_TILE // SC_C):
            pltpu.sync_copy(aggs.at[pl.ds(r0 + t * SC_C, SC_C)], nnv)
            pltpu.sync_copy(nnv, out_hbm.at[pl.ds(ob + t * SC_C, SC_C)])
        if rem:
            off = (ROWS_PER_TILE // SC_C) * SC_C
            pltpu.sync_copy(aggs.at[pl.ds(r0 + off, rem)], nnv.at[pl.ds(0, rem)])
            pltpu.sync_copy(nnv.at[pl.ds(0, rem)], out_hbm.at[pl.ds(ob + off, rem)])

        @pl.when(last)
        def _():
            toff = SC_NS * ROWS_PER_TILE
            tob = (g * SC_NC + cid) * N_NODES + toff
            pltpu.sync_copy(aggs.at[pl.ds(toff, ROWS_TAIL)],
                            nnv.at[pl.ds(0, ROWS_TAIL)])
            pltpu.sync_copy(nnv.at[pl.ds(0, ROWS_TAIL)],
                            out_hbm.at[pl.ds(tob, ROWS_TAIL)])

        plsc.subcore_barrier()
        return carry

    lax.fori_loop(0, N_GRAPHS, _per_graph, 0)


@functools.cache
def _get_sc_layer():
    return pl.kernel(
        _sc_body,
        mesh=plsc.VectorSubcoreMesh(core_axis_name="c", subcore_axis_name="s"),
        out_type=jax.ShapeDtypeStruct((N_GRAPHS * SC_NC * N_NODES, DIMP), jnp.float32),
        scratch_types=[
            pltpu.VMEM((SC_C,), jnp.int32),
            pltpu.VMEM((SC_C,), jnp.int32),
            pltpu.VMEM((SC_C, DIMP), jnp.float32),
            pltpu.VMEM((SC_C, DIM), jnp.float32),
            pltpu.VMEM((SC_C, DIMP), jnp.float32),
            pltpu.VMEM_SHARED((N_NODES, DIMP), jnp.float32),
            pltpu.SemaphoreType.DMA,
        ],
    )


def _sc_layer(nn_flat, h_flat, src9, dst):
    return _get_sc_layer()(nn_flat, h_flat, src9, dst)


def kernel(node_type, edge_index, edge_dist, graph_ids, fts, embed, W1, Wc1,
           bc1, Wc2, bc2, W2, b2, W3, b3, Wd1, bd1, Wd2, bd2, F1, bf1, F2,
           bf2, F3, bf3, F4, bf4):
    src = edge_index[0].astype(jnp.int32)
    dst = edge_index[1].astype(jnp.int32)
    src9 = (src[None, :] +
            (jnp.arange(N_GRAPHS, dtype=jnp.int32) * N_NODES)[:, None]).reshape(-1)

    nt4 = node_type.astype(jnp.int32).reshape(N_GRAPHS, N_NODES // NODE_CH, NODE_CH, 1)
    node, nn = _embed_call(nt4, embed, W1[0])

    dist4 = edge_dist.reshape(N_GRAPHS, N_EDGES // EDGE_CH, EDGE_CH, 1)
    hs = [_h_call(dist4, Wc1[l], bc1[l], Wc2[l], bc2[l]) for l in range(N_CONV)]

    for l in range(N_CONV):
        h_flat = hs[l].reshape(N_GRAPHS * N_EDGES, DIM)
        nn_flat = nn.reshape(N_GRAPHS * N_NODES, DIMP)
        agg = _sc_layer(nn_flat, h_flat, src9, dst)
        agg4 = agg.reshape(N_GRAPHS, SC_NC, N_NODES, DIMP)
        if l < N_CONV - 1:
            node, nn = _upd_call(node, agg4, W2[l], b2[l], W3[l], b3[l], W1[l + 1])
        else:
            node = _upd_call(node, agg4, W2[l], b2[l], W3[l], b3[l], None)

    gid3 = graph_ids.astype(jnp.int32).reshape(N_NODES // NODE_CH, NODE_CH, 1)
    res9 = _readout_call(node, gid3, Wd1, bd1, Wd2, bd2).reshape(N_GRAPHS, B)
    dense = jnp.concatenate([res9.T, fts], axis=1)
    return _mlp_call(dense, F1, bf1, F2, bf2, F3, bf3, F4, bf4)
